# baseline (speedup 1.0000x reference)
import functools

import jax
import jax.numpy as jnp
from jax import lax
from jax.experimental import pallas as pl
from jax.experimental.pallas import tpu as pltpu
from jax.experimental.pallas import tpu_sc as plsc

V = 1000000
EMB_DIM = 32
BATCH = 16384
NUM_NEG = 5
N_VN = BATCH * (1 + NUM_NEG)

NUM_CORES = 2
NUM_SUBCORES = 16
NUM_WORKERS = NUM_CORES * NUM_SUBCORES

NT = 7813
TPW = 244
WIN_T = 4
WIN = WIN_T * 128
NWIN = 62
V0T_MAX = (V - WIN) // 128
TAIL_V0 = NT // WIN_T * 0 + 999936
TAIL_W = 64

C_EFF = 4096
C_PAD = C_EFF + 32
W_PAD = C_EFF + 32 + 272

IDX_CHUNK = 4096
U_CHUNKS = BATCH // IDX_CHUNK
VN_CHUNKS = N_VN // IDX_CHUNK

OUT_U_ROWS = BATCH + 128
OUT_VN_ROWS = N_VN + 128

B_PER_W = BATCH // NUM_WORKERS
DOT_CHUNK = 128


def _i16(x):
  return jnp.full((16,), x, jnp.int32)


def _popcnt(mask):
  return jnp.max(plsc.all_reduce_population_count(mask))


def _sc_extract(pos_u, vn_idx, u3, v3):
  mesh = plsc.VectorSubcoreMesh(core_axis_name="c", subcore_axis_name="s")

  @functools.partial(
      pl.kernel,
      mesh=mesh,
      compiler_params=pltpu.CompilerParams(
          needs_layout_passes=False, use_tc_tiling_on_sc=True),
      out_type=[
          jax.ShapeDtypeStruct((OUT_U_ROWS, 128), jnp.float32),
          jax.ShapeDtypeStruct((OUT_VN_ROWS, 128), jnp.float32),
      ],
      scratch_types=[
          pltpu.VMEM((IDX_CHUNK,), jnp.int32),
          pltpu.VMEM((C_PAD,), jnp.int32),
          pltpu.VMEM((C_PAD,), jnp.int32),
          pltpu.VMEM((W_PAD,), jnp.int32),
          pltpu.VMEM((W_PAD,), jnp.int32),
          pltpu.VMEM((4, 8, WIN), jnp.float32),
          pltpu.VMEM((4, 8, WIN), jnp.float32),
          pltpu.VMEM((128, 128), jnp.float32),
          pltpu.VMEM((128, 128), jnp.float32),
          pltpu.VMEM((128,), jnp.int32),
          pltpu.VMEM((128,), jnp.int32),
          pltpu.SemaphoreType.DMA,
          pltpu.SemaphoreType.DMA,
          pltpu.SemaphoreType.DMA,
          pltpu.SemaphoreType.DMA,
      ],
  )
  def k(pu_hbm, vn_hbm, u3_hbm, v3_hbm, out_u, out_vn,
        idx_buf, lv, lp, wv, wp, wbuf0, wbuf1, stage0, stage1,
        sdst0, sdst1, wsem0, wsem1, ssem0, ssem1):
    wid = lax.axis_index("s") * NUM_CORES + lax.axis_index("c")
    base_t = wid * TPW + jnp.minimum(wid, 5)
    cnt_t = TPW + jnp.where(wid < 5, 1, 0)
    lo = base_t * 128
    hi = jnp.minimum((base_t + cnt_t) * 128, V)
    iota = lax.iota(jnp.int32, 16)

    wbufs = (wbuf0, wbuf1)
    wsems = (wsem0, wsem1)
    stages = (stage0, stage1)
    sdsts = (sdst0, sdst1)
    ssems = (ssem0, ssem1)

    def v0t_of(j):
      return jnp.minimum(base_t + WIN_T * j, V0T_MAX)

    def phase(idx_hbm, n_chunks, t3, out_hbm, dump_base):
      for b in (0, 1):
        for g in range(8):
          sdsts[b][pl.ds(g * 16, 16)] = _i16(dump_base) + g * 16 + iota
        pltpu.async_copy(stages[b], out_hbm.at[sdsts[b]], ssems[b])

      def scan(p):
        def chunk_body(ci, carry):
          pltpu.sync_copy(idx_hbm.at[pl.ds(ci * IDX_CHUNK, IDX_CHUNK)],
                          idx_buf)
          def vreg_body(i, carry2):
            lcount, cbefore = carry2
            vals = idx_buf[pl.ds(i * 16, 16)]
            m = (vals >= lo) & (vals < hi)
            npop = _popcnt(m)
            inwin = ((cbefore >= p * C_EFF)
                     & (cbefore < (p + 1) * C_EFF))
            keep = m & inwin
            plsc.store_compressed(lv.at[pl.ds(lcount, 16)], vals, mask=keep)
            pos = _i16(ci * IDX_CHUNK) + i * 16 + iota
            plsc.store_compressed(lp.at[pl.ds(lcount, 16)], pos, mask=keep)
            kpop = jnp.where(inwin, npop, 0)
            return (lcount + kpop, cbefore + npop)
          return lax.fori_loop(0, IDX_CHUNK // 16, vreg_body, carry)
        return lax.fori_loop(0, n_chunks, chunk_body, (0, 0))

      def fire(j, b):
        v0 = v0t_of(j) * 128
        for dhi in range(4):
          pltpu.async_copy(t3.at[dhi, :, pl.ds(v0, WIN)],
                           wbufs[b].at[dhi], wsems[b])

      def wait_win(j, b):
        v0 = v0t_of(j) * 128
        for dhi in range(4):
          pltpu.make_async_copy(t3.at[dhi, :, pl.ds(v0, WIN)],
                                wbufs[b].at[dhi], wsems[b]).wait()

      def process(buf, v0, width, lcount):
        def fb(i, wcount):
          lanem = (i * 16 + iota) < lcount
          vals = lv[pl.ds(i * 16, 16)]
          pos = lp[pl.ds(i * 16, 16)]
          m = lanem & (vals >= v0) & (vals < v0 + width)
          plsc.store_compressed(wv.at[pl.ds(wcount, 16)], vals - v0, mask=m)
          plsc.store_compressed(wp.at[pl.ds(wcount, 16)], pos, mask=m)
          return wcount + _popcnt(m)
        wcount = lax.fori_loop(0, (lcount + 15) // 16, fb, 0)

        def sg_body(s2, _):
          for b2 in (0, 1):
            off0 = s2 * 256 + b2 * 128
            pltpu.make_async_copy(stages[b2], out_hbm.at[sdsts[b2]],
                                  ssems[b2]).wait()
            for g in range(8):
              off = off0 + g * 16
              lanem = (off + iota) < wcount
              voff = jnp.where(lanem, wv[pl.ds(off, 16)], 0)
              dst = jnp.where(lanem, wp[pl.ds(off, 16)],
                              _i16(dump_base) + iota)
              sdsts[b2][pl.ds(g * 16, 16)] = dst
              def d_body(d, carry3):
                x = plsc.load_gather(
                    buf, [_i16(d // 8), _i16(d % 8), voff])
                plsc.store_scatter(
                    stages[b2], [_i16(g * 16) + iota, _i16(d)], x)
                return carry3
              lax.fori_loop(0, EMB_DIM, d_body, 0)
            pltpu.async_copy(stages[b2], out_hbm.at[sdsts[b2]], ssems[b2])
          return 0
        lax.fori_loop(0, (wcount + 255) // 256, sg_body, 0)

      def do_pass(p):
        lcount, total = scan(p)
        fire(0, 0)
        def win_body(j2, _):
          j = 2 * j2
          fire(j + 1, 1)
          wait_win(j, 0)
          process(wbufs[0], v0t_of(j) * 128, WIN, lcount)
          fire(j + 2, 0)
          wait_win(j + 1, 1)
          process(wbufs[1], v0t_of(j + 1) * 128, WIN, lcount)
          return 0
        lax.fori_loop(0, NWIN // 2, win_body, 0)
        wait_win(NWIN, 0)
        v0tail = base_t * 0 + TAIL_V0
        for dhi in range(4):
          pltpu.sync_copy(t3.at[dhi, :, pl.ds(v0tail, 128)],
                          wbufs[0].at[dhi, :, pl.ds(0, 128)])
        process(wbufs[0], v0tail, TAIL_W, lcount)
        return total

      total = do_pass(0)
      n_passes = (total + C_EFF - 1) // C_EFF
      lax.fori_loop(1, n_passes, lambda p, c: do_pass(p) * 0, 0)

      for b in (0, 1):
        pltpu.make_async_copy(stages[b], out_hbm.at[sdsts[b]],
                              ssems[b]).wait()

    phase(pu_hbm, U_CHUNKS, u3_hbm, out_u, BATCH)
    phase(vn_hbm, VN_CHUNKS, v3_hbm, out_vn, N_VN)

  return k(pos_u, vn_idx, u3, v3)


def _sc_dots(emb_u, emb_vn):
  mesh = plsc.VectorSubcoreMesh(core_axis_name="c", subcore_axis_name="s")

  @functools.partial(
      pl.kernel,
      mesh=mesh,
      compiler_params=pltpu.CompilerParams(
          needs_layout_passes=False, use_tc_tiling_on_sc=True),
      out_type=[
          jax.ShapeDtypeStruct((BATCH,), jnp.float32),
          jax.ShapeDtypeStruct((BATCH * NUM_NEG,), jnp.float32),
      ],
      scratch_types=[
          pltpu.VMEM((DOT_CHUNK, 128), jnp.float32),
          pltpu.VMEM((DOT_CHUNK, 128), jnp.float32),
          pltpu.VMEM((DOT_CHUNK * NUM_NEG, 128), jnp.float32),
          pltpu.VMEM((B_PER_W,), jnp.float32),
          pltpu.VMEM((B_PER_W * NUM_NEG,), jnp.float32),
          pltpu.SemaphoreType.DMA,
      ],
  )
  def k(u_hbm, vn_hbm, out_s, out_n, ub, vb, nb, s_out, n_out, sem):
    wid = lax.axis_index("s") * NUM_CORES + lax.axis_index("c")
    iota = lax.iota(jnp.int32, 16)
    for c in range(B_PER_W // DOT_CHUNK):
      row0 = wid * B_PER_W + c * DOT_CHUNK
      cps = [
          pltpu.async_copy(u_hbm.at[pl.ds(row0, DOT_CHUNK)], ub, sem),
          pltpu.async_copy(vn_hbm.at[pl.ds(row0, DOT_CHUNK)], vb, sem),
          pltpu.async_copy(
              vn_hbm.at[pl.ds(BATCH + row0 * NUM_NEG,
                              DOT_CHUNK * NUM_NEG)], nb, sem),
      ]
      for cp in cps:
        cp.wait()
      def group_body(g, carry):
        rows = g * 16 + iota
        acc_p = jnp.zeros((16,), jnp.float32)
        accs = [jnp.zeros((16,), jnp.float32) for _ in range(NUM_NEG)]
        nrows = rows * NUM_NEG
        for d in range(EMB_DIM):
          dvec = _i16(d)
          uu = plsc.load_gather(ub, [rows, dvec])
          vv = plsc.load_gather(vb, [rows, dvec])
          acc_p = acc_p + uu * vv
          for n in range(NUM_NEG):
            nn = plsc.load_gather(nb, [nrows + n, dvec])
            accs[n] = accs[n] + uu * nn
        s_out[pl.ds(c * DOT_CHUNK + g * 16, 16)] = acc_p
        for n in range(NUM_NEG):
          plsc.store_scatter(
              n_out, [_i16((c * DOT_CHUNK) * NUM_NEG + n) + nrows], accs[n])
        return carry
      lax.fori_loop(0, DOT_CHUNK // 16, group_body, 0)
    pltpu.sync_copy(s_out, out_s.at[pl.ds(wid * B_PER_W, B_PER_W)])
    pltpu.sync_copy(
        n_out, out_n.at[pl.ds(wid * B_PER_W * NUM_NEG, B_PER_W * NUM_NEG)])

  return k(emb_u, emb_vn)


def _softplus(x):
  return jnp.maximum(x, 0.0) + jnp.log(1.0 + jnp.exp(-jnp.abs(x)))


def _tc_loss_body(s_ref, n_ref, o_ref):
  s = jnp.clip(s_ref[...], -10.0, 10.0)
  n = jnp.clip(n_ref[...], -10.0, 10.0)
  total = jnp.sum(_softplus(-s)) + jnp.sum(_softplus(n))
  o_ref[...] = jnp.broadcast_to(total / BATCH, (1, 1))


def _tc_loss(score2d, neg2d):
  out = pl.pallas_call(
      _tc_loss_body,
      out_shape=jax.ShapeDtypeStruct((1, 1), jnp.float32),
  )(score2d, neg2d)
  return out[0, 0]


def kernel(pos_u, pos_v, neg_v, u_weight, v_weight):
  pos_u1d = pos_u.astype(jnp.int32).reshape(BATCH)
  vn1d = jnp.concatenate(
      [pos_v.astype(jnp.int32).reshape(BATCH),
       neg_v.astype(jnp.int32).reshape(BATCH * NUM_NEG)])
  u3 = u_weight.T.reshape(4, 8, V)
  v3 = v_weight.T.reshape(4, 8, V)
  emb_u, emb_vn = _sc_extract(pos_u1d, vn1d, u3, v3)
  score, negs = _sc_dots(emb_u, emb_vn)
  return _tc_loss(score.reshape(128, BATCH // 128),
                  negs.reshape(BATCH * NUM_NEG // 128, 128))


# speedup vs baseline: 2.6760x; 2.6760x over previous
import functools

import jax
import jax.numpy as jnp
from jax import lax
from jax.experimental import pallas as pl
from jax.experimental.pallas import tpu as pltpu
from jax.experimental.pallas import tpu_sc as plsc

V = 1000000
EMB_DIM = 32
BATCH = 16384
NUM_NEG = 5
N_VN = BATCH * (1 + NUM_NEG)

NUM_CORES = 2
NUM_SUBCORES = 16
NUM_WORKERS = NUM_CORES * NUM_SUBCORES

NT = 7813
TPW = 244
WIN_T = 4
WIN = WIN_T * 128
NWIN = 62
V0T_MAX = (V - WIN) // 128
TAIL_V0 = NT // WIN_T * 0 + 999936
TAIL_W = 64

C_EFF = 4096
C_PAD = C_EFF + 32
W_PAD = C_EFF + 32 + 272

IDX_CHUNK = 4096
U_CHUNKS = BATCH // IDX_CHUNK
VN_CHUNKS = N_VN // IDX_CHUNK

DUMP_SPREAD = 4096
OUT_U_ROWS = BATCH + DUMP_SPREAD
OUT_VN_ROWS = N_VN + DUMP_SPREAD

B_PER_W = BATCH // NUM_WORKERS
DOT_CHUNK = 128


def _i16(x):
  return jnp.full((16,), x, jnp.int32)


def _popcnt(mask):
  return jnp.max(plsc.all_reduce_population_count(mask))


def _sc_extract(pos_u, vn_idx, u3, v3):
  mesh = plsc.VectorSubcoreMesh(core_axis_name="c", subcore_axis_name="s")

  @functools.partial(
      pl.kernel,
      mesh=mesh,
      compiler_params=pltpu.CompilerParams(
          needs_layout_passes=False, use_tc_tiling_on_sc=True),
      out_type=[
          jax.ShapeDtypeStruct((OUT_U_ROWS, 128), jnp.float32),
          jax.ShapeDtypeStruct((OUT_VN_ROWS, 128), jnp.float32),
      ],
      scratch_types=[
          pltpu.VMEM((IDX_CHUNK,), jnp.int32),
          pltpu.VMEM((C_PAD,), jnp.int32),
          pltpu.VMEM((C_PAD,), jnp.int32),
          pltpu.VMEM((W_PAD,), jnp.int32),
          pltpu.VMEM((W_PAD,), jnp.int32),
          pltpu.VMEM((4, 8, WIN), jnp.float32),
          pltpu.VMEM((4, 8, WIN), jnp.float32),
          pltpu.VMEM((128, 128), jnp.float32),
          pltpu.VMEM((128, 128), jnp.float32),
          pltpu.VMEM((128,), jnp.int32),
          pltpu.VMEM((128,), jnp.int32),
          pltpu.SemaphoreType.DMA,
          pltpu.SemaphoreType.DMA,
          pltpu.SemaphoreType.DMA,
          pltpu.SemaphoreType.DMA,
      ],
  )
  def k(pu_hbm, vn_hbm, u3_hbm, v3_hbm, out_u, out_vn,
        idx_buf, lv, lp, wv, wp, wbuf0, wbuf1, stage0, stage1,
        sdst0, sdst1, wsem0, wsem1, ssem0, ssem1):
    wid = lax.axis_index("s") * NUM_CORES + lax.axis_index("c")
    base_t = wid * TPW + jnp.minimum(wid, 5)
    cnt_t = TPW + jnp.where(wid < 5, 1, 0)
    lo = base_t * 128
    hi = jnp.minimum((base_t + cnt_t) * 128, V)
    iota = lax.iota(jnp.int32, 16)

    wbufs = (wbuf0, wbuf1)
    wsems = (wsem0, wsem1)
    stages = (stage0, stage1)
    sdsts = (sdst0, sdst1)
    ssems = (ssem0, ssem1)

    def v0t_of(j):
      return jnp.minimum(base_t + WIN_T * j, V0T_MAX)

    def phase(idx_hbm, n_chunks, t3, out_hbm, dump_base):
      for b in (0, 1):
        for g in range(8):
          sdsts[b][pl.ds(g * 16, 16)] = (
              _i16(dump_base) + ((wid * 128 + g * 16 + iota) & (DUMP_SPREAD - 1)))
        pltpu.async_copy(stages[b], out_hbm.at[sdsts[b]], ssems[b])

      def scan(p):
        def chunk_body(ci, carry):
          pltpu.sync_copy(idx_hbm.at[pl.ds(ci * IDX_CHUNK, IDX_CHUNK)],
                          idx_buf)
          def vreg_body(i, carry2):
            lcount, cbefore = carry2
            vals = idx_buf[pl.ds(i * 16, 16)]
            m = (vals >= lo) & (vals < hi)
            npop = _popcnt(m)
            inwin = ((cbefore >= p * C_EFF)
                     & (cbefore < (p + 1) * C_EFF))
            keep = m & inwin
            plsc.store_compressed(lv.at[pl.ds(lcount, 16)], vals, mask=keep)
            pos = _i16(ci * IDX_CHUNK) + i * 16 + iota
            plsc.store_compressed(lp.at[pl.ds(lcount, 16)], pos, mask=keep)
            kpop = jnp.where(inwin, npop, 0)
            return (lcount + kpop, cbefore + npop)
          return lax.fori_loop(0, IDX_CHUNK // 16, vreg_body, carry)
        return lax.fori_loop(0, n_chunks, chunk_body, (0, 0))

      def fire(j, b):
        v0 = v0t_of(j) * 128
        for dhi in range(4):
          pltpu.async_copy(t3.at[dhi, :, pl.ds(v0, WIN)],
                           wbufs[b].at[dhi], wsems[b])

      def wait_win(j, b):
        v0 = v0t_of(j) * 128
        for dhi in range(4):
          pltpu.make_async_copy(t3.at[dhi, :, pl.ds(v0, WIN)],
                                wbufs[b].at[dhi], wsems[b]).wait()

      def process(buf, v0, width, lcount):
        def fb(i, wcount):
          lanem = (i * 16 + iota) < lcount
          vals = lv[pl.ds(i * 16, 16)]
          pos = lp[pl.ds(i * 16, 16)]
          m = lanem & (vals >= v0) & (vals < v0 + width)
          plsc.store_compressed(wv.at[pl.ds(wcount, 16)], vals - v0, mask=m)
          plsc.store_compressed(wp.at[pl.ds(wcount, 16)], pos, mask=m)
          return wcount + _popcnt(m)
        wcount = lax.fori_loop(0, (lcount + 15) // 16, fb, 0)

        def sg_body(s2, _):
          for b2 in (0, 1):
            off0 = s2 * 256 + b2 * 128
            pltpu.make_async_copy(stages[b2], out_hbm.at[sdsts[b2]],
                                  ssems[b2]).wait()
            for g in range(8):
              off = off0 + g * 16
              lanem = (off + iota) < wcount
              voff = jnp.where(lanem, wv[pl.ds(off, 16)], 0)
              dst = jnp.where(
                  lanem, wp[pl.ds(off, 16)],
                  _i16(dump_base) + ((wid * 128 + off + iota)
                                     & (DUMP_SPREAD - 1)))
              sdsts[b2][pl.ds(g * 16, 16)] = dst
              def d_body(d, carry3):
                x = plsc.load_gather(
                    buf, [_i16(d // 8), _i16(d % 8), voff])
                plsc.store_scatter(
                    stages[b2], [_i16(g * 16) + iota, _i16(d)], x)
                return carry3
              lax.fori_loop(0, EMB_DIM, d_body, 0)
            pltpu.async_copy(stages[b2], out_hbm.at[sdsts[b2]], ssems[b2])
          return 0
        lax.fori_loop(0, (wcount + 255) // 256, sg_body, 0)

      def do_pass(p):
        lcount, total = scan(p)
        fire(0, 0)
        def win_body(j2, _):
          j = 2 * j2
          fire(j + 1, 1)
          wait_win(j, 0)
          process(wbufs[0], v0t_of(j) * 128, WIN, lcount)
          fire(j + 2, 0)
          wait_win(j + 1, 1)
          process(wbufs[1], v0t_of(j + 1) * 128, WIN, lcount)
          return 0
        lax.fori_loop(0, NWIN // 2, win_body, 0)
        wait_win(NWIN, 0)
        v0tail = base_t * 0 + TAIL_V0
        for dhi in range(4):
          pltpu.sync_copy(t3.at[dhi, :, pl.ds(v0tail, 128)],
                          wbufs[0].at[dhi, :, pl.ds(0, 128)])
        process(wbufs[0], v0tail, TAIL_W, lcount)
        return total

      total = do_pass(0)
      n_passes = (total + C_EFF - 1) // C_EFF
      lax.fori_loop(1, n_passes, lambda p, c: do_pass(p) * 0, 0)

      for b in (0, 1):
        pltpu.make_async_copy(stages[b], out_hbm.at[sdsts[b]],
                              ssems[b]).wait()

    phase(pu_hbm, U_CHUNKS, u3_hbm, out_u, BATCH)
    phase(vn_hbm, VN_CHUNKS, v3_hbm, out_vn, N_VN)

  return k(pos_u, vn_idx, u3, v3)


def _sc_dots(emb_u, emb_vn):
  mesh = plsc.VectorSubcoreMesh(core_axis_name="c", subcore_axis_name="s")

  @functools.partial(
      pl.kernel,
      mesh=mesh,
      compiler_params=pltpu.CompilerParams(
          needs_layout_passes=False, use_tc_tiling_on_sc=True),
      out_type=[
          jax.ShapeDtypeStruct((BATCH,), jnp.float32),
          jax.ShapeDtypeStruct((BATCH * NUM_NEG,), jnp.float32),
      ],
      scratch_types=[
          pltpu.VMEM((DOT_CHUNK, 128), jnp.float32),
          pltpu.VMEM((DOT_CHUNK, 128), jnp.float32),
          pltpu.VMEM((DOT_CHUNK * NUM_NEG, 128), jnp.float32),
          pltpu.VMEM((B_PER_W,), jnp.float32),
          pltpu.VMEM((B_PER_W * NUM_NEG,), jnp.float32),
          pltpu.SemaphoreType.DMA,
      ],
  )
  def k(u_hbm, vn_hbm, out_s, out_n, ub, vb, nb, s_out, n_out, sem):
    wid = lax.axis_index("s") * NUM_CORES + lax.axis_index("c")
    iota = lax.iota(jnp.int32, 16)
    for c in range(B_PER_W // DOT_CHUNK):
      row0 = wid * B_PER_W + c * DOT_CHUNK
      cps = [
          pltpu.async_copy(u_hbm.at[pl.ds(row0, DOT_CHUNK)], ub, sem),
          pltpu.async_copy(vn_hbm.at[pl.ds(row0, DOT_CHUNK)], vb, sem),
          pltpu.async_copy(
              vn_hbm.at[pl.ds(BATCH + row0 * NUM_NEG,
                              DOT_CHUNK * NUM_NEG)], nb, sem),
      ]
      for cp in cps:
        cp.wait()
      def group_body(g, carry):
        rows = g * 16 + iota
        acc_p = jnp.zeros((16,), jnp.float32)
        accs = [jnp.zeros((16,), jnp.float32) for _ in range(NUM_NEG)]
        nrows = rows * NUM_NEG
        for d in range(EMB_DIM):
          dvec = _i16(d)
          uu = plsc.load_gather(ub, [rows, dvec])
          vv = plsc.load_gather(vb, [rows, dvec])
          acc_p = acc_p + uu * vv
          for n in range(NUM_NEG):
            nn = plsc.load_gather(nb, [nrows + n, dvec])
            accs[n] = accs[n] + uu * nn
        s_out[pl.ds(c * DOT_CHUNK + g * 16, 16)] = acc_p
        for n in range(NUM_NEG):
          plsc.store_scatter(
              n_out, [_i16((c * DOT_CHUNK) * NUM_NEG + n) + nrows], accs[n])
        return carry
      lax.fori_loop(0, DOT_CHUNK // 16, group_body, 0)
    pltpu.sync_copy(s_out, out_s.at[pl.ds(wid * B_PER_W, B_PER_W)])
    pltpu.sync_copy(
        n_out, out_n.at[pl.ds(wid * B_PER_W * NUM_NEG, B_PER_W * NUM_NEG)])

  return k(emb_u, emb_vn)


def _softplus(x):
  return jnp.maximum(x, 0.0) + jnp.log(1.0 + jnp.exp(-jnp.abs(x)))


def _tc_loss_body(s_ref, n_ref, o_ref):
  s = jnp.clip(s_ref[...], -10.0, 10.0)
  n = jnp.clip(n_ref[...], -10.0, 10.0)
  total = jnp.sum(_softplus(-s)) + jnp.sum(_softplus(n))
  o_ref[...] = jnp.broadcast_to(total / BATCH, (1, 1))


def _tc_loss(score2d, neg2d):
  out = pl.pallas_call(
      _tc_loss_body,
      out_shape=jax.ShapeDtypeStruct((1, 1), jnp.float32),
  )(score2d, neg2d)
  return out[0, 0]


def kernel(pos_u, pos_v, neg_v, u_weight, v_weight):
  pos_u1d = pos_u.astype(jnp.int32).reshape(BATCH)
  vn1d = jnp.concatenate(
      [pos_v.astype(jnp.int32).reshape(BATCH),
       neg_v.astype(jnp.int32).reshape(BATCH * NUM_NEG)])
  u3 = u_weight.T.reshape(4, 8, V)
  v3 = v_weight.T.reshape(4, 8, V)
  emb_u, emb_vn = _sc_extract(pos_u1d, vn1d, u3, v3)
  score, negs = _sc_dots(emb_u, emb_vn)
  return _tc_loss(score.reshape(128, BATCH // 128),
                  negs.reshape(BATCH * NUM_NEG // 128, 128))


# speedup vs baseline: 5.0517x; 1.8878x over previous
import functools

import jax
import jax.numpy as jnp
from jax import lax
from jax.experimental import pallas as pl
from jax.experimental.pallas import tpu as pltpu
from jax.experimental.pallas import tpu_sc as plsc

V = 1000000
EMB_DIM = 32
BATCH = 16384
NUM_NEG = 5
N_VN = BATCH * (1 + NUM_NEG)

NUM_CORES = 2
NUM_SUBCORES = 16
NUM_WORKERS = NUM_CORES * NUM_SUBCORES

NT = 7813
TPW = 244
WIN_T = 4
WIN = WIN_T * 128
NWIN = 62
V0T_MAX = (V - WIN) // 128
TAIL_V0 = NT // WIN_T * 0 + 999936
TAIL_W = 64

C_EFF = 4096
C_PAD = C_EFF + 32
W_PAD = C_EFF + 32 + 272

IDX_CHUNK = 4096
U_CHUNKS = BATCH // IDX_CHUNK
VN_CHUNKS = N_VN // IDX_CHUNK

DUMP_SPREAD = 4096
OUT_U_ROWS = BATCH + DUMP_SPREAD
OUT_VN_ROWS = N_VN + DUMP_SPREAD

B_PER_W = BATCH // NUM_WORKERS
DOT_CHUNK = 128


def _i16(x):
  return jnp.full((16,), x, jnp.int32)


def _popcnt(mask):
  return jnp.max(plsc.all_reduce_population_count(mask))


def _sc_extract(pos_u, vn_idx, u3, v3):
  mesh = plsc.VectorSubcoreMesh(core_axis_name="c", subcore_axis_name="s")

  @functools.partial(
      pl.kernel,
      mesh=mesh,
      compiler_params=pltpu.CompilerParams(
          needs_layout_passes=False, use_tc_tiling_on_sc=True),
      out_type=[
          jax.ShapeDtypeStruct((OUT_U_ROWS, 128), jnp.float32),
          jax.ShapeDtypeStruct((OUT_VN_ROWS, 128), jnp.float32),
      ],
      scratch_types=[
          pltpu.VMEM((IDX_CHUNK,), jnp.int32),
          pltpu.VMEM((C_PAD,), jnp.int32),
          pltpu.VMEM((C_PAD,), jnp.int32),
          pltpu.VMEM((W_PAD,), jnp.int32),
          pltpu.VMEM((W_PAD,), jnp.int32),
          pltpu.VMEM((4, 8, WIN), jnp.float32),
          pltpu.VMEM((4, 8, WIN), jnp.float32),
          pltpu.VMEM((128, 128), jnp.float32),
          pltpu.VMEM((128,), jnp.int32),
          pltpu.SemaphoreType.DMA,
          pltpu.SemaphoreType.DMA,
          pltpu.SemaphoreType.DMA,
      ],
  )
  def k(pu_hbm, vn_hbm, u3_hbm, v3_hbm, out_u, out_vn,
        idx_buf, lv, lp, wv, wp, wbuf0, wbuf1, stage0,
        sdst0, wsem0, wsem1, ssem0):
    wid = lax.axis_index("s") * NUM_CORES + lax.axis_index("c")
    base_t = wid * TPW + jnp.minimum(wid, 5)
    cnt_t = TPW + jnp.where(wid < 5, 1, 0)
    lo = base_t * 128
    hi = jnp.minimum((base_t + cnt_t) * 128, V)
    iota = lax.iota(jnp.int32, 16)

    wbufs = (wbuf0, wbuf1)
    wsems = (wsem0, wsem1)

    def v0t_of(j):
      return jnp.minimum(base_t + WIN_T * j, V0T_MAX)

    def phase(idx_hbm, n_chunks, t3, out_hbm, dump_base):
      for g in range(8):
        sdst0[pl.ds(g * 16, 16)] = (
            _i16(dump_base) + ((wid * 128 + g * 16 + iota) & (DUMP_SPREAD - 1)))

      def scan(p):
        def chunk_body(ci, carry):
          pltpu.sync_copy(idx_hbm.at[pl.ds(ci * IDX_CHUNK, IDX_CHUNK)],
                          idx_buf)
          def vreg_body(i, carry2):
            lcount, cbefore = carry2
            vals = idx_buf[pl.ds(i * 16, 16)]
            m = (vals >= lo) & (vals < hi)
            npop = _popcnt(m)
            inwin = ((cbefore >= p * C_EFF)
                     & (cbefore < (p + 1) * C_EFF))
            keep = m & inwin
            plsc.store_compressed(lv.at[pl.ds(lcount, 16)], vals, mask=keep)
            pos = _i16(ci * IDX_CHUNK) + i * 16 + iota
            plsc.store_compressed(lp.at[pl.ds(lcount, 16)], pos, mask=keep)
            kpop = jnp.where(inwin, npop, 0)
            return (lcount + kpop, cbefore + npop)
          return lax.fori_loop(0, IDX_CHUNK // 16, vreg_body, carry)
        return lax.fori_loop(0, n_chunks, chunk_body, (0, 0))

      def fire(j, b):
        v0 = v0t_of(j) * 128
        for dhi in range(4):
          pltpu.async_copy(t3.at[dhi, :, pl.ds(v0, WIN)],
                           wbufs[b].at[dhi], wsems[b])

      def wait_win(j, b):
        v0 = v0t_of(j) * 128
        for dhi in range(4):
          pltpu.make_async_copy(t3.at[dhi, :, pl.ds(v0, WIN)],
                                wbufs[b].at[dhi], wsems[b]).wait()

      def process(buf, v0, width, lcount, slot_in):
        def fb(i, wcount):
          lanem = (i * 16 + iota) < lcount
          vals = lv[pl.ds(i * 16, 16)]
          pos = lp[pl.ds(i * 16, 16)]
          m = lanem & (vals >= v0) & (vals < v0 + width)
          plsc.store_compressed(wv.at[pl.ds(wcount, 16)], vals - v0, mask=m)
          plsc.store_compressed(wp.at[pl.ds(wcount, 16)], pos, mask=m)
          return wcount + _popcnt(m)
        wcount = lax.fori_loop(0, (lcount + 15) // 16, fb, 0)

        def g_body(g, slot):
          off = g * 16
          lanem = (off + iota) < wcount
          voff = jnp.where(lanem, wv[pl.ds(off, 16)], 0)
          dst = jnp.where(
              lanem, wp[pl.ds(off, 16)],
              _i16(dump_base) + ((wid * 128 + off + iota)
                                 & (DUMP_SPREAD - 1)))
          row0 = slot * 16
          sdst0[pl.ds(row0, 16)] = dst
          def d_body(d, carry3):
            x = plsc.load_gather(
                buf, [_i16(d // 8), _i16(d % 8), voff])
            plsc.store_scatter(
                stage0, [_i16(row0) + iota, _i16(d)], x)
            return carry3
          lax.fori_loop(0, EMB_DIM, d_body, 0)
          def do_flush(_):
            pltpu.async_copy(stage0, out_hbm.at[sdst0], ssem0).wait()
            return 0
          lax.cond(slot == 7, do_flush, lambda _: 0, 0)
          return jnp.where(slot == 7, 0, slot + 1)
        return lax.fori_loop(0, (wcount + 15) // 16, g_body, slot_in)

      def do_pass(p, slot):
        lcount, total = scan(p)
        fire(0, 0)
        def win_body(j2, slot2):
          j = 2 * j2
          fire(j + 1, 1)
          wait_win(j, 0)
          slot2 = process(wbufs[0], v0t_of(j) * 128, WIN, lcount, slot2)
          fire(j + 2, 0)
          wait_win(j + 1, 1)
          slot2 = process(wbufs[1], v0t_of(j + 1) * 128, WIN, lcount, slot2)
          return slot2
        slot = lax.fori_loop(0, NWIN // 2, win_body, slot)
        wait_win(NWIN, 0)
        v0tail = base_t * 0 + TAIL_V0
        for dhi in range(4):
          pltpu.sync_copy(t3.at[dhi, :, pl.ds(v0tail, 128)],
                          wbufs[0].at[dhi, :, pl.ds(0, 128)])
        slot = process(wbufs[0], v0tail, TAIL_W, lcount, slot)
        return total, slot

      total, slot = do_pass(0, jnp.int32(0))
      n_passes = (total + C_EFF - 1) // C_EFF
      slot = lax.fori_loop(1, n_passes, lambda p, s: do_pass(p, s)[1], slot)

      pltpu.async_copy(stage0, out_hbm.at[sdst0], ssem0).wait()

    phase(pu_hbm, U_CHUNKS, u3_hbm, out_u, BATCH)
    phase(vn_hbm, VN_CHUNKS, v3_hbm, out_vn, N_VN)

  return k(pos_u, vn_idx, u3, v3)


def _sc_dots(emb_u, emb_vn):
  mesh = plsc.VectorSubcoreMesh(core_axis_name="c", subcore_axis_name="s")

  @functools.partial(
      pl.kernel,
      mesh=mesh,
      compiler_params=pltpu.CompilerParams(
          needs_layout_passes=False, use_tc_tiling_on_sc=True),
      out_type=[
          jax.ShapeDtypeStruct((BATCH,), jnp.float32),
          jax.ShapeDtypeStruct((BATCH * NUM_NEG,), jnp.float32),
      ],
      scratch_types=[
          pltpu.VMEM((DOT_CHUNK, 128), jnp.float32),
          pltpu.VMEM((DOT_CHUNK, 128), jnp.float32),
          pltpu.VMEM((DOT_CHUNK * NUM_NEG, 128), jnp.float32),
          pltpu.VMEM((B_PER_W,), jnp.float32),
          pltpu.VMEM((B_PER_W * NUM_NEG,), jnp.float32),
          pltpu.SemaphoreType.DMA,
      ],
  )
  def k(u_hbm, vn_hbm, out_s, out_n, ub, vb, nb, s_out, n_out, sem):
    wid = lax.axis_index("s") * NUM_CORES + lax.axis_index("c")
    iota = lax.iota(jnp.int32, 16)
    for c in range(B_PER_W // DOT_CHUNK):
      row0 = wid * B_PER_W + c * DOT_CHUNK
      cps = [
          pltpu.async_copy(u_hbm.at[pl.ds(row0, DOT_CHUNK)], ub, sem),
          pltpu.async_copy(vn_hbm.at[pl.ds(row0, DOT_CHUNK)], vb, sem),
          pltpu.async_copy(
              vn_hbm.at[pl.ds(BATCH + row0 * NUM_NEG,
                              DOT_CHUNK * NUM_NEG)], nb, sem),
      ]
      for cp in cps:
        cp.wait()
      def group_body(g, carry):
        rows = g * 16 + iota
        acc_p = jnp.zeros((16,), jnp.float32)
        accs = [jnp.zeros((16,), jnp.float32) for _ in range(NUM_NEG)]
        nrows = rows * NUM_NEG
        for d in range(EMB_DIM):
          dvec = _i16(d)
          uu = plsc.load_gather(ub, [rows, dvec])
          vv = plsc.load_gather(vb, [rows, dvec])
          acc_p = acc_p + uu * vv
          for n in range(NUM_NEG):
            nn = plsc.load_gather(nb, [nrows + n, dvec])
            accs[n] = accs[n] + uu * nn
        s_out[pl.ds(c * DOT_CHUNK + g * 16, 16)] = acc_p
        for n in range(NUM_NEG):
          plsc.store_scatter(
              n_out, [_i16((c * DOT_CHUNK) * NUM_NEG + n) + nrows], accs[n])
        return carry
      lax.fori_loop(0, DOT_CHUNK // 16, group_body, 0)
    pltpu.sync_copy(s_out, out_s.at[pl.ds(wid * B_PER_W, B_PER_W)])
    pltpu.sync_copy(
        n_out, out_n.at[pl.ds(wid * B_PER_W * NUM_NEG, B_PER_W * NUM_NEG)])

  return k(emb_u, emb_vn)


def _softplus(x):
  return jnp.maximum(x, 0.0) + jnp.log(1.0 + jnp.exp(-jnp.abs(x)))


def _tc_loss_body(s_ref, n_ref, o_ref):
  s = jnp.clip(s_ref[...], -10.0, 10.0)
  n = jnp.clip(n_ref[...], -10.0, 10.0)
  total = jnp.sum(_softplus(-s)) + jnp.sum(_softplus(n))
  o_ref[...] = jnp.broadcast_to(total / BATCH, (1, 1))


def _tc_loss(score2d, neg2d):
  out = pl.pallas_call(
      _tc_loss_body,
      out_shape=jax.ShapeDtypeStruct((1, 1), jnp.float32),
  )(score2d, neg2d)
  return out[0, 0]


def kernel(pos_u, pos_v, neg_v, u_weight, v_weight):
  pos_u1d = pos_u.astype(jnp.int32).reshape(BATCH)
  vn1d = jnp.concatenate(
      [pos_v.astype(jnp.int32).reshape(BATCH),
       neg_v.astype(jnp.int32).reshape(BATCH * NUM_NEG)])
  u3 = u_weight.T.reshape(4, 8, V)
  v3 = v_weight.T.reshape(4, 8, V)
  emb_u, emb_vn = _sc_extract(pos_u1d, vn1d, u3, v3)
  score, negs = _sc_dots(emb_u, emb_vn)
  return _tc_loss(score.reshape(128, BATCH // 128),
                  negs.reshape(BATCH * NUM_NEG // 128, 128))


# speedup vs baseline: 6.0471x; 1.1971x over previous
import functools

import jax
import jax.numpy as jnp
from jax import lax
from jax.experimental import pallas as pl
from jax.experimental.pallas import tpu as pltpu
from jax.experimental.pallas import tpu_sc as plsc

V = 1000000
EMB_DIM = 32
BATCH = 16384
NUM_NEG = 5
N_VN = BATCH * (1 + NUM_NEG)

NUM_CORES = 2
NUM_SUBCORES = 16
NUM_WORKERS = NUM_CORES * NUM_SUBCORES

NT = 7813
TPW = 244
WIN_T = 8
WIN = WIN_T * 128
NWIN = 32
V0T_MAX = (V - WIN) // 128
TAIL_V0 = NT // WIN_T * 0 + 999936
TAIL_W = 64

C_EFF = 4096
C_PAD = C_EFF + 32
W_PAD = C_EFF + 32 + 272

IDX_CHUNK = 8192
U_CHUNKS = BATCH // IDX_CHUNK
VN_CHUNKS = N_VN // IDX_CHUNK

DUMP_SPREAD = 4096
OUT_U_ROWS = BATCH + DUMP_SPREAD
OUT_VN_ROWS = N_VN + DUMP_SPREAD

B_PER_W = BATCH // NUM_WORKERS
DOT_CHUNK = 128


def _i16(x):
  return jnp.full((16,), x, jnp.int32)


def _popcnt(mask):
  return jnp.max(plsc.all_reduce_population_count(mask))


def _sc_extract(pos_u, vn_idx, u3, v3):
  mesh = plsc.VectorSubcoreMesh(core_axis_name="c", subcore_axis_name="s")

  @functools.partial(
      pl.kernel,
      mesh=mesh,
      compiler_params=pltpu.CompilerParams(
          needs_layout_passes=False, use_tc_tiling_on_sc=True),
      out_type=[
          jax.ShapeDtypeStruct((OUT_U_ROWS, 128), jnp.float32),
          jax.ShapeDtypeStruct((OUT_VN_ROWS, 128), jnp.float32),
      ],
      scratch_types=[
          pltpu.VMEM((IDX_CHUNK,), jnp.int32),
          pltpu.VMEM((C_PAD,), jnp.int32),
          pltpu.VMEM((C_PAD,), jnp.int32),
          pltpu.VMEM((W_PAD,), jnp.int32),
          pltpu.VMEM((W_PAD,), jnp.int32),
          pltpu.VMEM((4, 8, WIN), jnp.float32),
          pltpu.VMEM((4, 8, WIN), jnp.float32),
          pltpu.VMEM((128, 128), jnp.float32),
          pltpu.VMEM((128,), jnp.int32),
          pltpu.SemaphoreType.DMA,
          pltpu.SemaphoreType.DMA,
          pltpu.SemaphoreType.DMA,
      ],
  )
  def k(pu_hbm, vn_hbm, u3_hbm, v3_hbm, out_u, out_vn,
        idx_buf, lv, lp, wv, wp, wbuf0, wbuf1, stage0,
        sdst0, wsem0, wsem1, ssem0):
    wid = lax.axis_index("s") * NUM_CORES + lax.axis_index("c")
    base_t = wid * TPW + jnp.minimum(wid, 5)
    cnt_t = TPW + jnp.where(wid < 5, 1, 0)
    lo = base_t * 128
    hi = jnp.minimum((base_t + cnt_t) * 128, V)
    iota = lax.iota(jnp.int32, 16)

    wbufs = (wbuf0, wbuf1)
    wsems = (wsem0, wsem1)

    def v0t_of(j):
      return jnp.minimum(base_t + WIN_T * j, V0T_MAX)

    def phase(idx_hbm, n_chunks, t3, out_hbm, dump_base):
      for g in range(8):
        sdst0[pl.ds(g * 16, 16)] = (
            _i16(dump_base) + ((wid * 128 + g * 16 + iota) & (DUMP_SPREAD - 1)))

      def scan(p):
        def chunk_body(ci, carry):
          pltpu.sync_copy(idx_hbm.at[pl.ds(ci * IDX_CHUNK, IDX_CHUNK)],
                          idx_buf)
          def vreg_body(i, carry2):
            lcount, cbefore = carry2
            vals = idx_buf[pl.ds(i * 16, 16)]
            m = (vals >= lo) & (vals < hi)
            npop = _popcnt(m)
            inwin = ((cbefore >= p * C_EFF)
                     & (cbefore < (p + 1) * C_EFF))
            keep = m & inwin
            plsc.store_compressed(lv.at[pl.ds(lcount, 16)], vals, mask=keep)
            pos = _i16(ci * IDX_CHUNK) + i * 16 + iota
            plsc.store_compressed(lp.at[pl.ds(lcount, 16)], pos, mask=keep)
            kpop = jnp.where(inwin, npop, 0)
            return (lcount + kpop, cbefore + npop)
          return lax.fori_loop(0, IDX_CHUNK // 16, vreg_body, carry)
        return lax.fori_loop(0, n_chunks, chunk_body, (0, 0))

      def fire(j, b):
        v0 = v0t_of(j) * 128
        for dhi in range(4):
          pltpu.async_copy(t3.at[dhi, :, pl.ds(v0, WIN)],
                           wbufs[b].at[dhi], wsems[b])

      def wait_win(j, b):
        v0 = v0t_of(j) * 128
        for dhi in range(4):
          pltpu.make_async_copy(t3.at[dhi, :, pl.ds(v0, WIN)],
                                wbufs[b].at[dhi], wsems[b]).wait()

      def process(buf, v0, width, lcount, slot_in):
        def fb(i, wcount):
          lanem = (i * 16 + iota) < lcount
          vals = lv[pl.ds(i * 16, 16)]
          pos = lp[pl.ds(i * 16, 16)]
          m = lanem & (vals >= v0) & (vals < v0 + width)
          plsc.store_compressed(wv.at[pl.ds(wcount, 16)], vals - v0, mask=m)
          plsc.store_compressed(wp.at[pl.ds(wcount, 16)], pos, mask=m)
          return wcount + _popcnt(m)
        wcount = lax.fori_loop(0, (lcount + 15) // 16, fb, 0)

        def g_body(g, slot):
          off = g * 16
          lanem = (off + iota) < wcount
          voff = jnp.where(lanem, wv[pl.ds(off, 16)], 0)
          dst = jnp.where(
              lanem, wp[pl.ds(off, 16)],
              _i16(dump_base) + ((wid * 128 + off + iota)
                                 & (DUMP_SPREAD - 1)))
          row0 = slot * 16
          sdst0[pl.ds(row0, 16)] = dst
          def d_body(d, carry3):
            x = plsc.load_gather(
                buf, [_i16(d // 8), _i16(d % 8), voff])
            plsc.store_scatter(
                stage0, [_i16(row0) + iota, _i16(d)], x)
            return carry3
          lax.fori_loop(0, EMB_DIM, d_body, 0)
          def do_flush(_):
            pltpu.async_copy(stage0, out_hbm.at[sdst0], ssem0).wait()
            return 0
          lax.cond(slot == 7, do_flush, lambda _: 0, 0)
          return jnp.where(slot == 7, 0, slot + 1)
        return lax.fori_loop(0, (wcount + 15) // 16, g_body, slot_in)

      def do_pass(p, slot):
        lcount, total = scan(p)
        fire(0, 0)
        def win_body(j2, slot2):
          j = 2 * j2
          fire(j + 1, 1)
          wait_win(j, 0)
          slot2 = process(wbufs[0], v0t_of(j) * 128, WIN, lcount, slot2)
          fire(j + 2, 0)
          wait_win(j + 1, 1)
          slot2 = process(wbufs[1], v0t_of(j + 1) * 128, WIN, lcount, slot2)
          return slot2
        slot = lax.fori_loop(0, NWIN // 2, win_body, slot)
        wait_win(NWIN, 0)
        v0tail = base_t * 0 + TAIL_V0
        for dhi in range(4):
          pltpu.sync_copy(t3.at[dhi, :, pl.ds(v0tail, 128)],
                          wbufs[0].at[dhi, :, pl.ds(0, 128)])
        slot = process(wbufs[0], v0tail, TAIL_W, lcount, slot)
        return total, slot

      total, slot = do_pass(0, jnp.int32(0))
      n_passes = (total + C_EFF - 1) // C_EFF
      slot = lax.fori_loop(1, n_passes, lambda p, s: do_pass(p, s)[1], slot)

      pltpu.async_copy(stage0, out_hbm.at[sdst0], ssem0).wait()

    phase(pu_hbm, U_CHUNKS, u3_hbm, out_u, BATCH)
    phase(vn_hbm, VN_CHUNKS, v3_hbm, out_vn, N_VN)

  return k(pos_u, vn_idx, u3, v3)


def _sc_dots(emb_u, emb_vn):
  mesh = plsc.VectorSubcoreMesh(core_axis_name="c", subcore_axis_name="s")

  @functools.partial(
      pl.kernel,
      mesh=mesh,
      compiler_params=pltpu.CompilerParams(
          needs_layout_passes=False, use_tc_tiling_on_sc=True),
      out_type=[
          jax.ShapeDtypeStruct((BATCH,), jnp.float32),
          jax.ShapeDtypeStruct((BATCH * NUM_NEG,), jnp.float32),
      ],
      scratch_types=[
          pltpu.VMEM((DOT_CHUNK, 128), jnp.float32),
          pltpu.VMEM((DOT_CHUNK, 128), jnp.float32),
          pltpu.VMEM((DOT_CHUNK * NUM_NEG, 128), jnp.float32),
          pltpu.VMEM((B_PER_W,), jnp.float32),
          pltpu.VMEM((B_PER_W * NUM_NEG,), jnp.float32),
          pltpu.SemaphoreType.DMA,
      ],
  )
  def k(u_hbm, vn_hbm, out_s, out_n, ub, vb, nb, s_out, n_out, sem):
    wid = lax.axis_index("s") * NUM_CORES + lax.axis_index("c")
    iota = lax.iota(jnp.int32, 16)
    for c in range(B_PER_W // DOT_CHUNK):
      row0 = wid * B_PER_W + c * DOT_CHUNK
      cps = [
          pltpu.async_copy(u_hbm.at[pl.ds(row0, DOT_CHUNK)], ub, sem),
          pltpu.async_copy(vn_hbm.at[pl.ds(row0, DOT_CHUNK)], vb, sem),
          pltpu.async_copy(
              vn_hbm.at[pl.ds(BATCH + row0 * NUM_NEG,
                              DOT_CHUNK * NUM_NEG)], nb, sem),
      ]
      for cp in cps:
        cp.wait()
      def group_body(g, carry):
        rows = g * 16 + iota
        acc_p = jnp.zeros((16,), jnp.float32)
        accs = [jnp.zeros((16,), jnp.float32) for _ in range(NUM_NEG)]
        nrows = rows * NUM_NEG
        for d in range(EMB_DIM):
          dvec = _i16(d)
          uu = plsc.load_gather(ub, [rows, dvec])
          vv = plsc.load_gather(vb, [rows, dvec])
          acc_p = acc_p + uu * vv
          for n in range(NUM_NEG):
            nn = plsc.load_gather(nb, [nrows + n, dvec])
            accs[n] = accs[n] + uu * nn
        s_out[pl.ds(c * DOT_CHUNK + g * 16, 16)] = acc_p
        for n in range(NUM_NEG):
          plsc.store_scatter(
              n_out, [_i16((c * DOT_CHUNK) * NUM_NEG + n) + nrows], accs[n])
        return carry
      lax.fori_loop(0, DOT_CHUNK // 16, group_body, 0)
    pltpu.sync_copy(s_out, out_s.at[pl.ds(wid * B_PER_W, B_PER_W)])
    pltpu.sync_copy(
        n_out, out_n.at[pl.ds(wid * B_PER_W * NUM_NEG, B_PER_W * NUM_NEG)])

  return k(emb_u, emb_vn)


def _softplus(x):
  return jnp.maximum(x, 0.0) + jnp.log(1.0 + jnp.exp(-jnp.abs(x)))


def _tc_loss_body(s_ref, n_ref, o_ref):
  s = jnp.clip(s_ref[...], -10.0, 10.0)
  n = jnp.clip(n_ref[...], -10.0, 10.0)
  total = jnp.sum(_softplus(-s)) + jnp.sum(_softplus(n))
  o_ref[...] = jnp.broadcast_to(total / BATCH, (1, 1))


def _tc_loss(score2d, neg2d):
  out = pl.pallas_call(
      _tc_loss_body,
      out_shape=jax.ShapeDtypeStruct((1, 1), jnp.float32),
  )(score2d, neg2d)
  return out[0, 0]


def kernel(pos_u, pos_v, neg_v, u_weight, v_weight):
  pos_u1d = pos_u.astype(jnp.int32).reshape(BATCH)
  vn1d = jnp.concatenate(
      [pos_v.astype(jnp.int32).reshape(BATCH),
       neg_v.astype(jnp.int32).reshape(BATCH * NUM_NEG)])
  u3 = u_weight.T.reshape(4, 8, V)
  v3 = v_weight.T.reshape(4, 8, V)
  emb_u, emb_vn = _sc_extract(pos_u1d, vn1d, u3, v3)
  score, negs = _sc_dots(emb_u, emb_vn)
  return _tc_loss(score.reshape(128, BATCH // 128),
                  negs.reshape(BATCH * NUM_NEG // 128, 128))


# speedup vs baseline: 6.0609x; 1.0023x over previous
import functools

import jax
import jax.numpy as jnp
from jax import lax
from jax.experimental import pallas as pl
from jax.experimental.pallas import tpu as pltpu
from jax.experimental.pallas import tpu_sc as plsc

V = 1000000
EMB_DIM = 32
BATCH = 16384
NUM_NEG = 5
N_VN = BATCH * (1 + NUM_NEG)

NUM_CORES = 2
NUM_SUBCORES = 16
NUM_WORKERS = NUM_CORES * NUM_SUBCORES

NT = 7813
TPW = 244
WIN_T = 8
WIN = WIN_T * 128
NWIN = 32
V0T_MAX = (V - WIN) // 128
TAIL_V0 = NT // WIN_T * 0 + 999936
TAIL_W = 64

C_EFF = 4096
C_PAD = C_EFF + 32
W_PAD = C_EFF + 32 + 272

IDX_CHUNK = 8192
U_CHUNKS = BATCH // IDX_CHUNK
VN_CHUNKS = N_VN // IDX_CHUNK

DUMP_SPREAD = 4096
OUT_U_ROWS = BATCH + DUMP_SPREAD
OUT_VN_ROWS = N_VN + DUMP_SPREAD

B_PER_W = BATCH // NUM_WORKERS
DOT_CHUNK = 128


def _i16(x):
  return jnp.full((16,), x, jnp.int32)


def _popcnt(mask):
  return jnp.max(plsc.all_reduce_population_count(mask))


def _sc_extract(pos_u, vn_idx, u3, v3):
  mesh = plsc.VectorSubcoreMesh(core_axis_name="c", subcore_axis_name="s")

  @functools.partial(
      pl.kernel,
      mesh=mesh,
      compiler_params=pltpu.CompilerParams(
          needs_layout_passes=False, use_tc_tiling_on_sc=True),
      out_type=[
          jax.ShapeDtypeStruct((OUT_U_ROWS, 128), jnp.float32),
          jax.ShapeDtypeStruct((OUT_VN_ROWS, 128), jnp.float32),
      ],
      scratch_types=[
          pltpu.VMEM((IDX_CHUNK,), jnp.int32),
          pltpu.VMEM((C_PAD,), jnp.int32),
          pltpu.VMEM((C_PAD,), jnp.int32),
          pltpu.VMEM((W_PAD,), jnp.int32),
          pltpu.VMEM((W_PAD,), jnp.int32),
          pltpu.VMEM((4, 8, WIN), jnp.float32),
          pltpu.VMEM((4, 8, WIN), jnp.float32),
          pltpu.VMEM((128, 128), jnp.float32),
          pltpu.VMEM((128,), jnp.int32),
          pltpu.SemaphoreType.DMA,
          pltpu.SemaphoreType.DMA,
          pltpu.SemaphoreType.DMA,
      ],
  )
  def k(pu_hbm, vn_hbm, u3_hbm, v3_hbm, out_u, out_vn,
        idx_buf, lv, lp, wv, wp, wbuf0, wbuf1, stage0,
        sdst0, wsem0, wsem1, ssem0):
    wid = lax.axis_index("s") * NUM_CORES + lax.axis_index("c")
    base_t = wid * TPW + jnp.minimum(wid, 5)
    cnt_t = TPW + jnp.where(wid < 5, 1, 0)
    lo = base_t * 128
    hi = jnp.minimum((base_t + cnt_t) * 128, V)
    iota = lax.iota(jnp.int32, 16)

    wbufs = (wbuf0, wbuf1)
    wsems = (wsem0, wsem1)

    def v0t_of(j):
      return jnp.minimum(base_t + WIN_T * j, V0T_MAX)

    def phase(idx_hbm, n_chunks, t3, out_hbm, dump_base):
      for g in range(8):
        sdst0[pl.ds(g * 16, 16)] = (
            _i16(dump_base) + ((wid * 128 + g * 16 + iota) & (DUMP_SPREAD - 1)))

      def scan(p):
        def chunk_body(ci, carry):
          pltpu.sync_copy(idx_hbm.at[pl.ds(ci * IDX_CHUNK, IDX_CHUNK)],
                          idx_buf)
          def vreg_body(i, carry2):
            lcount, cbefore = carry2
            vals = idx_buf[pl.ds(i * 16, 16)]
            m = (vals >= lo) & (vals < hi)
            npop = _popcnt(m)
            inwin = ((cbefore >= p * C_EFF)
                     & (cbefore < (p + 1) * C_EFF))
            keep = m & inwin
            plsc.store_compressed(lv.at[pl.ds(lcount, 16)], vals, mask=keep)
            pos = _i16(ci * IDX_CHUNK) + i * 16 + iota
            plsc.store_compressed(lp.at[pl.ds(lcount, 16)], pos, mask=keep)
            kpop = jnp.where(inwin, npop, 0)
            return (lcount + kpop, cbefore + npop)
          return lax.fori_loop(0, IDX_CHUNK // 16, vreg_body, carry)
        return lax.fori_loop(0, n_chunks, chunk_body, (0, 0))

      def fire(j, b):
        v0 = v0t_of(j) * 128
        pltpu.async_copy(t3.at[:, :, pl.ds(v0, WIN)], wbufs[b], wsems[b])

      def wait_win(j, b):
        v0 = v0t_of(j) * 128
        pltpu.make_async_copy(t3.at[:, :, pl.ds(v0, WIN)],
                              wbufs[b], wsems[b]).wait()

      def process(buf, v0, width, lcount, slot_in):
        def fb(i, wcount):
          lanem = (i * 16 + iota) < lcount
          vals = lv[pl.ds(i * 16, 16)]
          pos = lp[pl.ds(i * 16, 16)]
          m = lanem & (vals >= v0) & (vals < v0 + width)
          plsc.store_compressed(wv.at[pl.ds(wcount, 16)], vals - v0, mask=m)
          plsc.store_compressed(wp.at[pl.ds(wcount, 16)], pos, mask=m)
          return wcount + _popcnt(m)
        wcount = lax.fori_loop(0, (lcount + 15) // 16, fb, 0)

        def g_body(g, slot):
          off = g * 16
          lanem = (off + iota) < wcount
          voff = jnp.where(lanem, wv[pl.ds(off, 16)], 0)
          dst = jnp.where(
              lanem, wp[pl.ds(off, 16)],
              _i16(dump_base) + ((wid * 128 + off + iota)
                                 & (DUMP_SPREAD - 1)))
          row0 = slot * 16
          sdst0[pl.ds(row0, 16)] = dst
          def d_body(d, carry3):
            x = plsc.load_gather(
                buf, [_i16(d // 8), _i16(d % 8), voff])
            plsc.store_scatter(
                stage0, [_i16(row0) + iota, _i16(d)], x)
            return carry3
          lax.fori_loop(0, EMB_DIM, d_body, 0)
          def do_flush(_):
            pltpu.async_copy(stage0, out_hbm.at[sdst0], ssem0).wait()
            return 0
          lax.cond(slot == 7, do_flush, lambda _: 0, 0)
          return jnp.where(slot == 7, 0, slot + 1)
        return lax.fori_loop(0, (wcount + 15) // 16, g_body, slot_in)

      def do_pass(p, slot):
        lcount, total = scan(p)
        fire(0, 0)
        def win_body(j2, slot2):
          j = 2 * j2
          fire(j + 1, 1)
          wait_win(j, 0)
          slot2 = process(wbufs[0], v0t_of(j) * 128, WIN, lcount, slot2)
          fire(j + 2, 0)
          wait_win(j + 1, 1)
          slot2 = process(wbufs[1], v0t_of(j + 1) * 128, WIN, lcount, slot2)
          return slot2
        slot = lax.fori_loop(0, NWIN // 2, win_body, slot)
        wait_win(NWIN, 0)
        v0tail = base_t * 0 + TAIL_V0
        for dhi in range(4):
          pltpu.sync_copy(t3.at[dhi, :, pl.ds(v0tail, 128)],
                          wbufs[0].at[dhi, :, pl.ds(0, 128)])
        slot = process(wbufs[0], v0tail, TAIL_W, lcount, slot)
        return total, slot

      total, slot = do_pass(0, jnp.int32(0))
      n_passes = (total + C_EFF - 1) // C_EFF
      slot = lax.fori_loop(1, n_passes, lambda p, s: do_pass(p, s)[1], slot)

      pltpu.async_copy(stage0, out_hbm.at[sdst0], ssem0).wait()

    phase(pu_hbm, U_CHUNKS, u3_hbm, out_u, BATCH)
    phase(vn_hbm, VN_CHUNKS, v3_hbm, out_vn, N_VN)

  return k(pos_u, vn_idx, u3, v3)


def _sc_dots(emb_u, emb_vn):
  mesh = plsc.VectorSubcoreMesh(core_axis_name="c", subcore_axis_name="s")

  @functools.partial(
      pl.kernel,
      mesh=mesh,
      compiler_params=pltpu.CompilerParams(
          needs_layout_passes=False, use_tc_tiling_on_sc=True),
      out_type=[
          jax.ShapeDtypeStruct((BATCH,), jnp.float32),
          jax.ShapeDtypeStruct((BATCH * NUM_NEG,), jnp.float32),
      ],
      scratch_types=[
          pltpu.VMEM((DOT_CHUNK, 128), jnp.float32),
          pltpu.VMEM((DOT_CHUNK, 128), jnp.float32),
          pltpu.VMEM((DOT_CHUNK * NUM_NEG, 128), jnp.float32),
          pltpu.VMEM((B_PER_W,), jnp.float32),
          pltpu.VMEM((B_PER_W * NUM_NEG,), jnp.float32),
          pltpu.SemaphoreType.DMA,
      ],
  )
  def k(u_hbm, vn_hbm, out_s, out_n, ub, vb, nb, s_out, n_out, sem):
    wid = lax.axis_index("s") * NUM_CORES + lax.axis_index("c")
    iota = lax.iota(jnp.int32, 16)
    for c in range(B_PER_W // DOT_CHUNK):
      row0 = wid * B_PER_W + c * DOT_CHUNK
      cps = [
          pltpu.async_copy(u_hbm.at[pl.ds(row0, DOT_CHUNK)], ub, sem),
          pltpu.async_copy(vn_hbm.at[pl.ds(row0, DOT_CHUNK)], vb, sem),
          pltpu.async_copy(
              vn_hbm.at[pl.ds(BATCH + row0 * NUM_NEG,
                              DOT_CHUNK * NUM_NEG)], nb, sem),
      ]
      for cp in cps:
        cp.wait()
      def group_body(g, carry):
        rows = g * 16 + iota
        acc_p = jnp.zeros((16,), jnp.float32)
        accs = [jnp.zeros((16,), jnp.float32) for _ in range(NUM_NEG)]
        nrows = rows * NUM_NEG
        for d in range(EMB_DIM):
          dvec = _i16(d)
          uu = plsc.load_gather(ub, [rows, dvec])
          vv = plsc.load_gather(vb, [rows, dvec])
          acc_p = acc_p + uu * vv
          for n in range(NUM_NEG):
            nn = plsc.load_gather(nb, [nrows + n, dvec])
            accs[n] = accs[n] + uu * nn
        s_out[pl.ds(c * DOT_CHUNK + g * 16, 16)] = acc_p
        for n in range(NUM_NEG):
          plsc.store_scatter(
              n_out, [_i16((c * DOT_CHUNK) * NUM_NEG + n) + nrows], accs[n])
        return carry
      lax.fori_loop(0, DOT_CHUNK // 16, group_body, 0)
    pltpu.sync_copy(s_out, out_s.at[pl.ds(wid * B_PER_W, B_PER_W)])
    pltpu.sync_copy(
        n_out, out_n.at[pl.ds(wid * B_PER_W * NUM_NEG, B_PER_W * NUM_NEG)])

  return k(emb_u, emb_vn)


def _softplus(x):
  return jnp.maximum(x, 0.0) + jnp.log(1.0 + jnp.exp(-jnp.abs(x)))


def _tc_loss_body(s_ref, n_ref, o_ref):
  s = jnp.clip(s_ref[...], -10.0, 10.0)
  n = jnp.clip(n_ref[...], -10.0, 10.0)
  total = jnp.sum(_softplus(-s)) + jnp.sum(_softplus(n))
  o_ref[...] = jnp.broadcast_to(total / BATCH, (1, 1))


def _tc_loss(score2d, neg2d):
  out = pl.pallas_call(
      _tc_loss_body,
      out_shape=jax.ShapeDtypeStruct((1, 1), jnp.float32),
  )(score2d, neg2d)
  return out[0, 0]


def kernel(pos_u, pos_v, neg_v, u_weight, v_weight):
  pos_u1d = pos_u.astype(jnp.int32).reshape(BATCH)
  vn1d = jnp.concatenate(
      [pos_v.astype(jnp.int32).reshape(BATCH),
       neg_v.astype(jnp.int32).reshape(BATCH * NUM_NEG)])
  u3 = u_weight.T.reshape(4, 8, V)
  v3 = v_weight.T.reshape(4, 8, V)
  emb_u, emb_vn = _sc_extract(pos_u1d, vn1d, u3, v3)
  score, negs = _sc_dots(emb_u, emb_vn)
  return _tc_loss(score.reshape(128, BATCH // 128),
                  negs.reshape(BATCH * NUM_NEG // 128, 128))
